# Initial kernel scaffold; baseline (speedup 1.0000x reference)
import jax
import jax.numpy as jnp
from jax.experimental import pallas as pl


def kernel(x, edge_index, W1, b1, W2, b2, Wm, bm):
    raise NotImplementedError("write your pallas kernel here")


# speedup vs baseline: 12.9579x; 12.9579x over previous
import functools

import jax
import jax.numpy as jnp
from jax import lax
from jax.experimental import pallas as pl
from jax.experimental.pallas import tpu as pltpu
from jax.experimental.pallas import tpu_sc as plsc

NC = 2
NS = 16
NW = NC * NS
B = 128

_mesh = plsc.VectorSubcoreMesh(core_axis_name="c", subcore_axis_name="s")


def _ceil_to(a, m):
  return -(-a // m) * m


def _make_deg_kernel(n_acc, k):
  @functools.partial(
      pl.kernel,
      out_type=jax.ShapeDtypeStruct((NC, n_acc, 16), jnp.float32),
      mesh=_mesh,
      scratch_types=[
          pltpu.VMEM((k, B), jnp.int32),
          pltpu.VMEM((B, 16), jnp.float32),
          pltpu.VMEM((B, 16), jnp.float32),
          pltpu.VMEM_SHARED((n_acc, 16), jnp.float32),
      ],
  )
  def deg_kernel(col_hbm, ones_hbm, zeros_hbm, out_hbm, col_v, ones_v, zbuf_v,
                 acc_sh):
    cid = lax.axis_index("c")
    sid = lax.axis_index("s")
    wid = cid * NS + sid
    seg = n_acc // NS
    nchunk = seg // B

    pltpu.sync_copy(zeros_hbm, zbuf_v)
    def zbody(j, _):
      pltpu.sync_copy(zbuf_v, acc_sh.at[pl.ds(sid * seg + j * B, B)])
      return 0
    lax.fori_loop(0, nchunk, zbody, 0)
    pltpu.sync_copy(ones_hbm, ones_v)
    pltpu.sync_copy(col_hbm.at[wid], col_v)
    plsc.subcore_barrier()

    def body(j, _):
      pltpu.sync_copy(ones_v, acc_sh.at[col_v.at[j]], add=True)
      return 0
    lax.fori_loop(0, k, body, 0)
    plsc.subcore_barrier()

    def obody(j, _):
      r0 = sid * seg + j * B
      pltpu.sync_copy(acc_sh.at[pl.ds(r0, B)], zbuf_v)
      pltpu.sync_copy(zbuf_v, out_hbm.at[cid, pl.ds(r0, B)])
      return 0
    lax.fori_loop(0, nchunk, obody, 0)

  return deg_kernel


def _make_aggr_kernel(n, n_acc, k):
  @functools.partial(
      pl.kernel,
      out_type=jax.ShapeDtypeStruct((NC, n_acc, 128), jnp.float32),
      mesh=_mesh,
      scratch_types=[
          pltpu.VMEM((k, B), jnp.int32),
          pltpu.VMEM((k, B), jnp.int32),
          pltpu.VMEM((B, 128), jnp.float32),
          pltpu.VMEM_SHARED((n_acc, 128), jnp.float32),
          pltpu.SemaphoreType.DMA,
      ],
  )
  def aggr_kernel(y_hbm, pk_hbm, zeros_hbm, out_hbm,
                  row_v, col_v, fbuf_v, acc_sh, sem):
    cid = lax.axis_index("c")
    sid = lax.axis_index("s")
    wid = cid * NS + sid
    seg = n_acc // NS
    nchunk = seg // B

    pltpu.sync_copy(zeros_hbm, fbuf_v)
    def zbody(j, _):
      pltpu.sync_copy(fbuf_v, acc_sh.at[pl.ds(sid * seg + j * B, B)])
      return 0
    lax.fori_loop(0, nchunk, zbody, 0)
    pltpu.sync_copy(pk_hbm.at[wid], row_v)

    def ubody(i, _):
      r = i // 8
      c = (i % 8) * 16
      v = row_v[r, pl.ds(c, 16)]
      col_v[r, pl.ds(c, 16)] = lax.bitwise_and(v, 0xFFFF)
      row_v[r, pl.ds(c, 16)] = lax.shift_right_logical(v, 16)
      return 0
    lax.fori_loop(0, k * 8, ubody, 0)
    plsc.subcore_barrier()

    def body(j, _):
      pltpu.async_copy(y_hbm.at[row_v.at[j]], fbuf_v, sem).wait()
      pltpu.sync_copy(fbuf_v, acc_sh.at[col_v.at[j]], add=True)
      return 0
    lax.fori_loop(0, k, body, 0)
    plsc.subcore_barrier()

    def obody(j, _):
      r0 = sid * seg + j * B
      pltpu.sync_copy(acc_sh.at[pl.ds(r0, B)], fbuf_v)
      pltpu.sync_copy(fbuf_v, out_hbm.at[cid, pl.ds(r0, B)])
      return 0
    lax.fori_loop(0, nchunk, obody, 0)

  return aggr_kernel


def _dis_from_parts(dp):
  deg = dp[0, :, 0:1] + dp[1, :, 0:1] + 1.0
  return lax.rsqrt(deg)


def _stage1_body(x_ref, w1_ref, dp_ref, y_ref, xw_ref):
  dis = _dis_from_parts(dp_ref[...])
  xw = jnp.dot(x_ref[...], w1_ref[...], preferred_element_type=jnp.float32)
  xw_ref[...] = xw
  y_ref[...] = xw * dis


def _stage2_body(s_ref, xw_ref, dp_ref, w_ref, b_ref, y_ref, xw2_ref):
  dis = _dis_from_parts(dp_ref[...])
  s = s_ref[0] + s_ref[1]
  h = jnp.maximum(dis * s + (dis * dis) * xw_ref[...] + b_ref[...], 0.0)
  xw2 = jnp.dot(h, w_ref[...], preferred_element_type=jnp.float32)
  xw2_ref[...] = xw2
  y_ref[...] = xw2 * dis


def _stage3_body(s_ref, xw_ref, dp_ref, b_ref, wm_ref, bm_ref, o_ref):
  dis = _dis_from_parts(dp_ref[...])
  s = s_ref[0] + s_ref[1]
  h = jnp.maximum(dis * s + (dis * dis) * xw_ref[...] + b_ref[...], 0.0)
  o_ref[...] = jnp.dot(h, wm_ref[...],
                       preferred_element_type=jnp.float32) + bm_ref[...]


def _node_spec(bn):
  return pl.BlockSpec((bn, 128), lambda i: (i, 0))


def _part_spec(bn, w):
  return pl.BlockSpec((NC, bn, w), lambda i: (0, i, 0))


def _full_spec(shape):
  return pl.BlockSpec(shape, lambda i: tuple(0 for _ in shape))


def kernel(x, edge_index, W1, b1, W2, b2, Wm, bm):
  n, d = x.shape
  e = edge_index.shape[1]
  k = -(-e // (NW * B))
  e_pad = NW * B * k
  n_acc = _ceil_to(n + 1, NS * B)

  row = jnp.concatenate(
      [edge_index[0], jnp.zeros((e_pad - e,), jnp.int32)]).reshape(NW, k, B)
  col = jnp.concatenate(
      [edge_index[1], jnp.full((e_pad - e,), n, jnp.int32)]).reshape(NW, k, B)
  packed = jnp.bitwise_or(jnp.left_shift(row, 16), col)

  ones16 = jnp.ones((B, 16), jnp.float32)
  zeros16 = jnp.zeros((B, 16), jnp.float32)
  zeros128 = jnp.zeros((B, 128), jnp.float32)
  b1r = b1.reshape(1, 128)
  b2r = b2.reshape(1, 128)
  bmr = bm.reshape(1, 128)

  deg_parts = _make_deg_kernel(n_acc, k)(col, ones16, zeros16)
  aggr = _make_aggr_kernel(n, n_acc, k)

  bn = 1000
  grid = n // bn

  y1, xw1 = pl.pallas_call(
      _stage1_body,
      grid=(grid,),
      in_specs=[_node_spec(bn), _full_spec((128, 128)), _part_spec(bn, 16)],
      out_specs=[_node_spec(bn), _node_spec(bn)],
      out_shape=[jax.ShapeDtypeStruct((n, 128), jnp.float32)] * 2,
  )(x, W1, deg_parts)

  s1 = aggr(y1, packed, zeros128)

  y2, xw2 = pl.pallas_call(
      _stage2_body,
      grid=(grid,),
      in_specs=[_part_spec(bn, 128), _node_spec(bn), _part_spec(bn, 16),
                _full_spec((128, 128)), _full_spec((1, 128))],
      out_specs=[_node_spec(bn), _node_spec(bn)],
      out_shape=[jax.ShapeDtypeStruct((n, 128), jnp.float32)] * 2,
  )(s1, xw1, deg_parts, W2, b1r)

  s2 = aggr(y2, packed, zeros128)

  out = pl.pallas_call(
      _stage3_body,
      grid=(grid,),
      in_specs=[_part_spec(bn, 128), _node_spec(bn), _part_spec(bn, 16),
                _full_spec((1, 128)), _full_spec((128, 128)),
                _full_spec((1, 128))],
      out_specs=_node_spec(bn),
      out_shape=jax.ShapeDtypeStruct((n, 128), jnp.float32),
  )(s2, xw2, deg_parts, b2r, Wm, bmr)

  return out
